# baseline (speedup 1.0000x reference)
import jax
import jax.numpy as jnp
from jax.experimental import pallas as pl
from jax.experimental.pallas import tpu as pltpu

_BN = 128


def _sage_body(a_ref, x_ref, wl_ref, bl_ref, wr_ref, o_ref, xhi_s, xlo_s):
    @pl.when(pl.program_id(0) == 0)
    def _split_x():
        x = x_ref[...]
        x_hi = x.astype(jnp.bfloat16)
        xhi_s[...] = x_hi
        xlo_s[...] = (x - x_hi.astype(jnp.float32)).astype(jnp.bfloat16)

    a = a_ref[...].astype(jnp.bfloat16)
    dn = (((0,), (0,)), ((), ()))
    num = (jax.lax.dot_general(a, xhi_s[...], dn, preferred_element_type=jnp.float32)
           + jax.lax.dot_general(a, xlo_s[...], dn, preferred_element_type=jnp.float32))
    ones = jnp.ones((a.shape[0], 1), dtype=jnp.bfloat16)
    cnt = jax.lax.dot_general(a, ones, dn, preferred_element_type=jnp.float32)
    agg = num / jnp.maximum(cnt, 1.0)
    dt = (((1,), (1,)), ((), ()))
    h = jax.lax.dot_general(
        agg, wl_ref[...], dt,
        preferred_element_type=jnp.float32,
        precision=jax.lax.Precision.HIGHEST)
    j = pl.program_id(0)
    x_root = x_ref[pl.ds(j * _BN, _BN), :]
    h = h + bl_ref[...]
    h = h + jax.lax.dot_general(
        x_root, wr_ref[...], dt,
        preferred_element_type=jnp.float32,
        precision=jax.lax.Precision.HIGHEST)
    o_ref[...] = h


def kernel(features, adjacency_matrix, W_l, b_l, W_r):
    n, d = features.shape
    grid = (n // _BN,)
    return pl.pallas_call(
        _sage_body,
        grid=grid,
        in_specs=[
            pl.BlockSpec((n, _BN), lambda j: (0, j)),
            pl.BlockSpec((n, d), lambda j: (0, 0)),
            pl.BlockSpec((d, d), lambda j: (0, 0)),
            pl.BlockSpec((1, d), lambda j: (0, 0)),
            pl.BlockSpec((d, d), lambda j: (0, 0)),
        ],
        out_specs=pl.BlockSpec((_BN, d), lambda j: (j, 0)),
        out_shape=jax.ShapeDtypeStruct((n, d), jnp.float32),
        scratch_shapes=[
            pltpu.VMEM((n, d), jnp.bfloat16),
            pltpu.VMEM((n, d), jnp.bfloat16),
        ],
        compiler_params=pltpu.CompilerParams(
            dimension_semantics=("arbitrary",)),
    )(adjacency_matrix, features, W_l, b_l.reshape(1, d), W_r)


# speedup vs baseline: 1.5791x; 1.5791x over previous
import jax
import jax.numpy as jnp
from jax.experimental import pallas as pl
from jax.experimental.pallas import tpu as pltpu

_NK = 4


def _sage_body(a_ref, x_ref, wl_ref, bl_ref, wr_ref, o_ref, num_s, cnt_s):
    k = pl.program_id(0)
    bk = a_ref.shape[0]
    a = a_ref[...].astype(jnp.bfloat16)
    x_blk = x_ref[pl.ds(k * bk, bk), :]
    x_hi = x_blk.astype(jnp.bfloat16)
    x_lo = (x_blk - x_hi.astype(jnp.float32)).astype(jnp.bfloat16)
    dn = (((0,), (0,)), ((), ()))
    num_p = (jax.lax.dot_general(a, x_hi, dn, preferred_element_type=jnp.float32)
             + jax.lax.dot_general(a, x_lo, dn, preferred_element_type=jnp.float32))
    ones = jnp.ones((bk, 1), dtype=jnp.bfloat16)
    cnt_p = jax.lax.dot_general(a, ones, dn, preferred_element_type=jnp.float32)

    @pl.when(k == 0)
    def _init():
        num_s[...] = num_p
        cnt_s[...] = cnt_p

    @pl.when(k > 0)
    def _accum():
        num_s[...] += num_p
        cnt_s[...] += cnt_p

    @pl.when(k == _NK - 1)
    def _epilogue():
        agg = num_s[...] / jnp.maximum(cnt_s[...], 1.0)
        dt = (((1,), (1,)), ((), ()))
        h = jax.lax.dot_general(
            agg, wl_ref[...], dt,
            preferred_element_type=jnp.float32,
            precision=jax.lax.Precision.HIGHEST)
        h = h + bl_ref[...]
        h = h + jax.lax.dot_general(
            x_ref[...], wr_ref[...], dt,
            preferred_element_type=jnp.float32,
            precision=jax.lax.Precision.HIGHEST)
        o_ref[...] = h


def kernel(features, adjacency_matrix, W_l, b_l, W_r):
    n, d = features.shape
    bk = n // _NK
    return pl.pallas_call(
        _sage_body,
        grid=(_NK,),
        in_specs=[
            pl.BlockSpec((bk, n), lambda k: (k, 0)),
            pl.BlockSpec((n, d), lambda k: (0, 0)),
            pl.BlockSpec((d, d), lambda k: (0, 0)),
            pl.BlockSpec((1, d), lambda k: (0, 0)),
            pl.BlockSpec((d, d), lambda k: (0, 0)),
        ],
        out_specs=pl.BlockSpec((n, d), lambda k: (0, 0)),
        out_shape=jax.ShapeDtypeStruct((n, d), jnp.float32),
        scratch_shapes=[
            pltpu.VMEM((n, d), jnp.float32),
            pltpu.VMEM((n, 1), jnp.float32),
        ],
        compiler_params=pltpu.CompilerParams(
            dimension_semantics=("arbitrary",)),
    )(adjacency_matrix, features, W_l, b_l.reshape(1, d), W_r)
